# baseline (speedup 1.0000x reference)
import jax
import jax.numpy as jnp
from jax import lax
from jax.experimental import pallas as pl
from jax.experimental.pallas import tpu as pltpu

BT = 128


def _linear_kernel(x_ref, w_ref, b_ref, o_ref):
    nb, bt, d = x_ref.shape
    l = w_ref.shape[0]
    xm = x_ref[...].reshape(nb * bt, d)
    r = lax.dot_general(
        w_ref[...], xm,
        dimension_numbers=(((1,), (1,)), ((), ())),
        preferred_element_type=jnp.float32,
    )
    r = r + b_ref[...].T
    o_ref[...] = r.reshape(l, nb, bt)


def kernel(encoding, W, b):
    B, T, D = encoding.shape
    L = W.shape[0]
    b2 = b.reshape(1, L)

    out = pl.pallas_call(
        _linear_kernel,
        grid=(T // BT,),
        in_specs=[
            pl.BlockSpec((B, BT, D), lambda i: (0, i, 0)),
            pl.BlockSpec((L, D), lambda i: (0, 0)),
            pl.BlockSpec((1, L), lambda i: (0, 0)),
        ],
        out_specs=pl.BlockSpec((L, B, BT), lambda i: (0, 0, i)),
        out_shape=jax.ShapeDtypeStruct((L, B, T), jnp.float32),
        compiler_params=pltpu.CompilerParams(
            dimension_semantics=("parallel",),
        ),
    )(encoding, W, b2)
    return out.transpose(1, 2, 0)


# speedup vs baseline: 1.1957x; 1.1957x over previous
import jax
import jax.numpy as jnp
from jax import lax
from jax.experimental import pallas as pl
from jax.experimental.pallas import tpu as pltpu

BT = 256


def _linear_kernel(x_ref, w_ref, b_ref, o_ref):
    nb, bt, d = x_ref.shape
    l = w_ref.shape[0]
    xm = x_ref[...].reshape(nb * bt, d)
    r = lax.dot_general(
        w_ref[...], xm,
        dimension_numbers=(((1,), (1,)), ((), ())),
        preferred_element_type=jnp.float32,
    )
    r = r + b_ref[...].T
    o_ref[...] = r.reshape(l, nb, bt)


def kernel(encoding, W, b):
    B, T, D = encoding.shape
    L = W.shape[0]
    b2 = b.reshape(1, L)

    out = pl.pallas_call(
        _linear_kernel,
        grid=(T // BT,),
        in_specs=[
            pl.BlockSpec((B, BT, D), lambda i: (0, i, 0)),
            pl.BlockSpec((L, D), lambda i: (0, 0)),
            pl.BlockSpec((1, L), lambda i: (0, 0)),
        ],
        out_specs=pl.BlockSpec((L, B, BT), lambda i: (0, 0, i)),
        out_shape=jax.ShapeDtypeStruct((L, B, T), jnp.float32),
        compiler_params=pltpu.CompilerParams(
            dimension_semantics=("parallel",),
        ),
    )(encoding, W, b2)
    return out.transpose(1, 2, 0)
